# baseline (device time: 270946 ns/iter reference)
import jax
import jax.numpy as jnp
from jax import lax
from jax.experimental import pallas as pl
from jax.experimental.pallas import tpu as pltpu

N_DEV = 4
SQ = 2048
SKV = 2048
D_MODEL = 1024
H_LOC = 8
DH = 128
BLK = 64
SCALE = 0.08838834764831843


QT = 512
N_QT = SQ // QT


def _compute_body(x_ref, wq_ref, k_ref, v_ref, wo_ref, out_ref, s_ref, ctx_ref):
    t = pl.program_id(0)
    h = pl.program_id(1)
    rows = pl.ds(t * QT, QT)
    hcols = pl.ds(h * DH, DH)
    q = jnp.dot(
        x_ref[rows, :], wq_ref[:, hcols], preferred_element_type=jnp.float32
    )

    for kt in range(N_QT):
        sl = slice(kt * QT, (kt + 1) * QT)

        @pl.when(kt <= t)
        def _():
            s = lax.dot_general(
                q, k_ref[h, sl, :], (((1,), (1,)), ((), ())),
                preferred_element_type=jnp.float32,
            ) * SCALE
            row = t * QT + lax.broadcasted_iota(jnp.int32, (QT, QT), 0)
            col = kt * QT + lax.broadcasted_iota(jnp.int32, (QT, QT), 1)
            s_ref[:, sl] = jnp.where((col // BLK) <= (row // BLK), s, -1e9)

        @pl.when(kt > t)
        def _():
            s_ref[:, sl] = jnp.full((QT, QT), -1e9, jnp.float32)

    sfull = s_ref[...]
    m = jnp.max(sfull, axis=-1, keepdims=True)
    w = jnp.exp(sfull - m)
    w = w / jnp.sum(w, axis=-1, keepdims=True)

    ctx_ref[...] = jnp.zeros((QT, DH), jnp.float32)
    for kt in range(N_QT):
        sl = slice(kt * QT, (kt + 1) * QT)

        @pl.when(kt <= t)
        def _():
            ctx_ref[...] += jnp.dot(
                w[:, sl], v_ref[h, sl, :], preferred_element_type=jnp.float32
            )

    contrib = jnp.dot(
        ctx_ref[...], wo_ref[hcols, :], preferred_element_type=jnp.float32
    )

    @pl.when(h == 0)
    def _():
        out_ref[rows, :] = contrib

    @pl.when(h != 0)
    def _():
        out_ref[rows, :] += contrib


CHUNK = SQ // (2 * N_DEV)


def _allreduce_body(p_ref, out_ref, comm_ref, send_sems, recv_sems):
    my = lax.axis_index("i")
    left = lax.rem(my + (N_DEV - 1), N_DEV)
    right = lax.rem(my + 1, N_DEV)

    def mod4(v):
        return lax.rem(v + 4 * N_DEV, N_DEV)

    def rows_r(c):
        return c * CHUNK

    def rows_l(c):
        return N_DEV * CHUNK + c * CHUNK

    barrier_sem = pltpu.get_barrier_semaphore()
    for nbr in (left, right):
        pl.semaphore_signal(
            barrier_sem, inc=1,
            device_id=(nbr,), device_id_type=pl.DeviceIdType.MESH,
        )
    pl.semaphore_wait(barrier_sem, 2)

    out_ref[...] = p_ref[...]

    def copy(src_start, dst_start, dst_is_out, dev, sem_idx):
        dst = out_ref if dst_is_out else comm_ref
        return pltpu.make_async_remote_copy(
            src_ref=out_ref.at[pl.ds(src_start, CHUNK), :],
            dst_ref=dst.at[pl.ds(dst_start, CHUNK), :],
            send_sem=send_sems.at[sem_idx],
            recv_sem=recv_sems.at[sem_idx],
            device_id=(dev,),
            device_id_type=pl.DeviceIdType.MESH,
        )

    for s in range(N_DEV - 1):
        r_send = copy(rows_r(mod4(my - s)), s * CHUNK, False, right, s)
        l_send = copy(rows_l(mod4(my + s)), (3 + s) * CHUNK, False, left, 3 + s)
        r_send.start()
        l_send.start()
        r_send.wait()
        l_send.wait()
        rr = rows_r(mod4(my - s - 1))
        rl = rows_l(mod4(my + s + 1))
        out_ref[pl.ds(rr, CHUNK), :] += comm_ref[pl.ds(s * CHUNK, CHUNK), :]
        out_ref[pl.ds(rl, CHUNK), :] += comm_ref[pl.ds((3 + s) * CHUNK, CHUNK), :]

    for s in range(N_DEV - 1):
        cr = rows_r(mod4(my + 1 - s))
        cl = rows_l(mod4(my - 1 + s))
        r_send = copy(cr, cr, True, right, 6 + s)
        l_send = copy(cl, cl, True, left, 9 + s)
        r_send.start()
        l_send.start()
        r_send.wait()
        l_send.wait()


def kernel(x, Wq, K_ext, V_ext, Wo):
    i = lax.axis_index("i")
    Wq_loc = lax.dynamic_slice(Wq, (0, i * (H_LOC * DH)), (D_MODEL, H_LOC * DH))
    Wo_loc = lax.dynamic_slice(Wo, (i * (H_LOC * DH), 0), (H_LOC * DH, D_MODEL))
    x2 = x.reshape(SQ, D_MODEL)
    K = K_ext.reshape(SKV, H_LOC, DH).transpose(1, 0, 2)
    V = V_ext.reshape(SKV, H_LOC, DH).transpose(1, 0, 2)

    partial = pl.pallas_call(
        _compute_body,
        grid=(N_QT, H_LOC),
        in_specs=[
            pl.BlockSpec((SQ, D_MODEL), lambda t, h: (0, 0)),
            pl.BlockSpec((D_MODEL, H_LOC * DH), lambda t, h: (0, 0)),
            pl.BlockSpec((H_LOC, SKV, DH), lambda t, h: (0, 0, 0)),
            pl.BlockSpec((H_LOC, SKV, DH), lambda t, h: (0, 0, 0)),
            pl.BlockSpec((H_LOC * DH, D_MODEL), lambda t, h: (0, 0)),
        ],
        out_specs=pl.BlockSpec((SQ, D_MODEL), lambda t, h: (0, 0)),
        out_shape=jax.ShapeDtypeStruct((SQ, D_MODEL), jnp.float32),
        scratch_shapes=[
            pltpu.VMEM((QT, SKV), jnp.float32),
            pltpu.VMEM((QT, DH), jnp.float32),
        ],
        compiler_params=pltpu.CompilerParams(
            vmem_limit_bytes=100 * 1024 * 1024
        ),
    )(x2, Wq_loc, K, V, Wo_loc)

    out = pl.pallas_call(
        _allreduce_body,
        out_shape=jax.ShapeDtypeStruct((SQ, D_MODEL), jnp.float32),
        in_specs=[pl.BlockSpec(memory_space=pltpu.VMEM)],
        out_specs=pl.BlockSpec(memory_space=pltpu.VMEM),
        scratch_shapes=[
            pltpu.VMEM((6 * CHUNK, D_MODEL), jnp.float32),
            pltpu.SemaphoreType.DMA((12,)),
            pltpu.SemaphoreType.DMA((12,)),
        ],
        compiler_params=pltpu.CompilerParams(
            collective_id=0, vmem_limit_bytes=100 * 1024 * 1024
        ),
    )(partial)

    return out.reshape(1, SQ, D_MODEL)


# device time: 209228 ns/iter; 1.2950x vs baseline; 1.2950x over previous
import jax
import jax.numpy as jnp
from jax import lax
from jax.experimental import pallas as pl
from jax.experimental.pallas import tpu as pltpu

N_DEV = 4
SQ = 2048
SKV = 2048
D_MODEL = 1024
H_LOC = 8
DH = 128
BLK = 64
SCALE = 0.08838834764831843
CH = SQ // (2 * N_DEV)


def _fused_body(
    x_hbm, wq_hbm, k_hbm, v_hbm, wo_hbm, out_ref,
    xv, wqv, qv, kv, vv, wov, comm_ref,
    load_sems, send_sems, recv_sems,
):
    my = lax.axis_index("i")
    left = lax.rem(my + (N_DEV - 1), N_DEV)
    right = lax.rem(my + 1, N_DEV)

    def mod4(v):
        return lax.rem(v + 4 * N_DEV, N_DEV)

    def r_rows(c):
        return c * CH

    def l_rows(c):
        return N_DEV * CH + c * CH

    barrier_sem = pltpu.get_barrier_semaphore()
    for nbr in (left, right):
        pl.semaphore_signal(
            barrier_sem, inc=1,
            device_id=(nbr,), device_id_type=pl.DeviceIdType.MESH,
        )
    pl.semaphore_wait(barrier_sem, 2)

    loads = [
        pltpu.make_async_copy(x_hbm, xv, load_sems.at[0]),
        pltpu.make_async_copy(wq_hbm, wqv, load_sems.at[1]),
        pltpu.make_async_copy(k_hbm, kv, load_sems.at[2]),
        pltpu.make_async_copy(v_hbm, vv, load_sems.at[3]),
        pltpu.make_async_copy(wo_hbm, wov, load_sems.at[4]),
    ]
    for cp in loads:
        cp.start()
    loads[0].wait()
    loads[1].wait()
    qv[...] = jnp.dot(xv[...], wqv[...], preferred_element_type=jnp.float32)
    loads[2].wait()
    loads[3].wait()
    loads[4].wait()

    def compute_tile(row_start):
        for h in range(H_LOC):
            hcols = slice(h * DH, (h + 1) * DH)
            q = qv[pl.ds(row_start, CH), hcols]
            s = lax.dot_general(
                q, kv[h], (((1,), (1,)), ((), ())),
                preferred_element_type=jnp.float32,
            ) * SCALE
            rowg = row_start + lax.broadcasted_iota(jnp.int32, (CH, SKV), 0)
            col = lax.broadcasted_iota(jnp.int32, (CH, SKV), 1)
            s = jnp.where((col // BLK) <= (rowg // BLK), s, -1e9)
            m = jnp.max(s, axis=-1, keepdims=True)
            w = jnp.exp(s - m)
            w = w / jnp.sum(w, axis=-1, keepdims=True)
            ctx = jnp.dot(w, vv[h], preferred_element_type=jnp.float32)
            contrib = jnp.dot(
                ctx, wov[hcols, :], preferred_element_type=jnp.float32
            )
            if h == 0:
                out_ref[pl.ds(row_start, CH), :] = contrib
            else:
                out_ref[pl.ds(row_start, CH), :] += contrib

    def rs_copy(src_start, slot, dev, sem_idx):
        return pltpu.make_async_remote_copy(
            src_ref=out_ref.at[pl.ds(src_start, CH), :],
            dst_ref=comm_ref.at[pl.ds(slot * CH, CH), :],
            send_sem=send_sems.at[sem_idx],
            recv_sem=recv_sems.at[sem_idx],
            device_id=(dev,),
            device_id_type=pl.DeviceIdType.MESH,
        )

    def ag_copy(row_start, dev, sem_idx):
        return pltpu.make_async_remote_copy(
            src_ref=out_ref.at[pl.ds(row_start, CH), :],
            dst_ref=out_ref.at[pl.ds(row_start, CH), :],
            send_sem=send_sems.at[sem_idx],
            recv_sem=recv_sems.at[sem_idx],
            device_id=(dev,),
            device_id_type=pl.DeviceIdType.MESH,
        )

    def rs_absorb(rdma, row_start, slot):
        rdma.wait()
        out_ref[pl.ds(row_start, CH), :] += comm_ref[pl.ds(slot * CH, CH), :]

    rs_r = [None] * 3
    rs_l = [None] * 3
    compute_tile(r_rows(my))
    rs_r[0] = rs_copy(r_rows(my), 0, right, 0)
    rs_r[0].start()
    compute_tile(l_rows(my))
    rs_l[0] = rs_copy(l_rows(my), 3, left, 3)
    rs_l[0].start()
    for s in range(1, 3):
        cr = mod4(my - s)
        compute_tile(r_rows(cr))
        rs_absorb(rs_r[s - 1], r_rows(cr), s - 1)
        rs_r[s] = rs_copy(r_rows(cr), s, right, s)
        rs_r[s].start()
        cl = mod4(my + s)
        compute_tile(l_rows(cl))
        rs_absorb(rs_l[s - 1], l_rows(cl), 3 + s - 1)
        rs_l[s] = rs_copy(l_rows(cl), 3 + s, left, 3 + s)
        rs_l[s].start()
    compute_tile(r_rows(mod4(my + 1)))
    rs_absorb(rs_r[2], r_rows(mod4(my + 1)), 2)
    ag_r0 = ag_copy(r_rows(mod4(my + 1)), right, 6)
    ag_r0.start()
    compute_tile(l_rows(mod4(my - 1)))
    rs_absorb(rs_l[2], l_rows(mod4(my - 1)), 5)
    ag_l0 = ag_copy(l_rows(mod4(my - 1)), left, 9)
    ag_l0.start()

    ag_r0.wait()
    ag_r1 = ag_copy(r_rows(my), right, 7)
    ag_r1.start()
    ag_l0.wait()
    ag_l1 = ag_copy(l_rows(my), left, 10)
    ag_l1.start()
    ag_r1.wait()
    ag_r2 = ag_copy(r_rows(mod4(my - 1)), right, 8)
    ag_r2.start()
    ag_l1.wait()
    ag_l2 = ag_copy(l_rows(mod4(my + 1)), left, 11)
    ag_l2.start()
    ag_r2.wait()
    ag_l2.wait()


def kernel(x, Wq, K_ext, V_ext, Wo):
    i = lax.axis_index("i")
    Wq_loc = lax.dynamic_slice(Wq, (0, i * (H_LOC * DH)), (D_MODEL, H_LOC * DH))
    Wo_loc = lax.dynamic_slice(Wo, (i * (H_LOC * DH), 0), (H_LOC * DH, D_MODEL))
    x2 = x.reshape(SQ, D_MODEL)
    K = K_ext.reshape(SKV, H_LOC, DH).transpose(1, 0, 2)
    V = V_ext.reshape(SKV, H_LOC, DH).transpose(1, 0, 2)

    out = pl.pallas_call(
        _fused_body,
        out_shape=jax.ShapeDtypeStruct((SQ, D_MODEL), jnp.float32),
        in_specs=[pl.BlockSpec(memory_space=pl.ANY)] * 5,
        out_specs=pl.BlockSpec(memory_space=pltpu.VMEM),
        scratch_shapes=[
            pltpu.VMEM((SQ, D_MODEL), jnp.float32),
            pltpu.VMEM((D_MODEL, H_LOC * DH), jnp.float32),
            pltpu.VMEM((SQ, H_LOC * DH), jnp.float32),
            pltpu.VMEM((H_LOC, SKV, DH), jnp.float32),
            pltpu.VMEM((H_LOC, SKV, DH), jnp.float32),
            pltpu.VMEM((H_LOC * DH, D_MODEL), jnp.float32),
            pltpu.VMEM((6 * CH, D_MODEL), jnp.float32),
            pltpu.SemaphoreType.DMA((5,)),
            pltpu.SemaphoreType.DMA((12,)),
            pltpu.SemaphoreType.DMA((12,)),
        ],
        compiler_params=pltpu.CompilerParams(
            collective_id=0, vmem_limit_bytes=100 * 1024 * 1024
        ),
    )(x2, Wq_loc, K, V, Wo_loc)

    return out.reshape(1, SQ, D_MODEL)


# device time: 206311 ns/iter; 1.3133x vs baseline; 1.0141x over previous
import jax
import jax.numpy as jnp
from jax import lax
from jax.experimental import pallas as pl
from jax.experimental.pallas import tpu as pltpu

N_DEV = 4
SQ = 2048
SKV = 2048
D_MODEL = 1024
H_LOC = 8
DH = 128
BLK = 64
SCALE = 0.08838834764831843
QT = 512
N_QT = SQ // QT
CHUNK = SQ // (2 * N_DEV)


def _compute_body(x_ref, wq_ref, k_ref, v_ref, wo_ref, out_ref):
    h = pl.program_id(0)
    q = jnp.dot(x_ref[...], wq_ref[...], preferred_element_type=jnp.float32)
    k = k_ref[0]
    v = v_ref[0]
    ctxs = []
    for qt in range(N_QT):
        qrows = slice(qt * QT, (qt + 1) * QT)
        kl = (qt + 1) * QT
        s = lax.dot_general(
            q[qrows, :], k[:kl, :], (((1,), (1,)), ((), ())),
            preferred_element_type=jnp.float32,
        ) * SCALE
        row = qt * QT + lax.broadcasted_iota(jnp.int32, (QT, kl), 0)
        col = lax.broadcasted_iota(jnp.int32, (QT, kl), 1)
        s = jnp.where((col // BLK) <= (row // BLK), s, -1e9)
        m = jnp.max(s, axis=-1, keepdims=True)
        w = jnp.exp(s - m)
        w = w / jnp.sum(w, axis=-1, keepdims=True)
        ctxs.append(
            jnp.dot(w, v[:kl, :], preferred_element_type=jnp.float32)
        )
    ctx = jnp.concatenate(ctxs, axis=0)
    contrib = jnp.dot(ctx, wo_ref[...], preferred_element_type=jnp.float32)

    @pl.when(h == 0)
    def _():
        out_ref[...] = contrib

    @pl.when(h != 0)
    def _():
        out_ref[...] += contrib


def _allreduce_body(p_ref, out_ref, comm_ref, send_sems, recv_sems):
    my = lax.axis_index("i")
    left = lax.rem(my + (N_DEV - 1), N_DEV)
    right = lax.rem(my + 1, N_DEV)

    def mod4(v):
        return lax.rem(v + 4 * N_DEV, N_DEV)

    def rows_r(c):
        return c * CHUNK

    def rows_l(c):
        return N_DEV * CHUNK + c * CHUNK

    barrier_sem = pltpu.get_barrier_semaphore()
    for nbr in (left, right):
        pl.semaphore_signal(
            barrier_sem, inc=1,
            device_id=(nbr,), device_id_type=pl.DeviceIdType.MESH,
        )
    pl.semaphore_wait(barrier_sem, 2)

    out_ref[...] = p_ref[...]

    def copy(src_start, dst_start, dst_is_out, dev, sem_idx):
        dst = out_ref if dst_is_out else comm_ref
        return pltpu.make_async_remote_copy(
            src_ref=out_ref.at[pl.ds(src_start, CHUNK), :],
            dst_ref=dst.at[pl.ds(dst_start, CHUNK), :],
            send_sem=send_sems.at[sem_idx],
            recv_sem=recv_sems.at[sem_idx],
            device_id=(dev,),
            device_id_type=pl.DeviceIdType.MESH,
        )

    for s in range(N_DEV - 1):
        r_send = copy(rows_r(mod4(my - s)), s * CHUNK, False, right, s)
        l_send = copy(rows_l(mod4(my + s)), (3 + s) * CHUNK, False, left, 3 + s)
        r_send.start()
        l_send.start()
        r_send.wait()
        l_send.wait()
        rr = rows_r(mod4(my - s - 1))
        rl = rows_l(mod4(my + s + 1))
        out_ref[pl.ds(rr, CHUNK), :] += comm_ref[pl.ds(s * CHUNK, CHUNK), :]
        out_ref[pl.ds(rl, CHUNK), :] += comm_ref[pl.ds((3 + s) * CHUNK, CHUNK), :]

    for s in range(N_DEV - 1):
        cr = rows_r(mod4(my + 1 - s))
        cl = rows_l(mod4(my - 1 + s))
        r_send = copy(cr, cr, True, right, 6 + s)
        l_send = copy(cl, cl, True, left, 9 + s)
        r_send.start()
        l_send.start()
        r_send.wait()
        l_send.wait()


def kernel(x, Wq, K_ext, V_ext, Wo):
    i = lax.axis_index("i")
    Wq_loc = lax.dynamic_slice(Wq, (0, i * (H_LOC * DH)), (D_MODEL, H_LOC * DH))
    Wo_loc = lax.dynamic_slice(Wo, (i * (H_LOC * DH), 0), (H_LOC * DH, D_MODEL))
    x2 = x.reshape(SQ, D_MODEL)
    K = K_ext.reshape(SKV, H_LOC, DH).transpose(1, 0, 2)
    V = V_ext.reshape(SKV, H_LOC, DH).transpose(1, 0, 2)

    partial = pl.pallas_call(
        _compute_body,
        grid=(H_LOC,),
        in_specs=[
            pl.BlockSpec((SQ, D_MODEL), lambda h: (0, 0)),
            pl.BlockSpec((D_MODEL, DH), lambda h: (0, h)),
            pl.BlockSpec((1, SKV, DH), lambda h: (h, 0, 0)),
            pl.BlockSpec((1, SKV, DH), lambda h: (h, 0, 0)),
            pl.BlockSpec((DH, D_MODEL), lambda h: (h, 0)),
        ],
        out_specs=pl.BlockSpec((SQ, D_MODEL), lambda h: (0, 0)),
        out_shape=jax.ShapeDtypeStruct((SQ, D_MODEL), jnp.float32),
        compiler_params=pltpu.CompilerParams(
            vmem_limit_bytes=100 * 1024 * 1024
        ),
    )(x2, Wq_loc, K, V, Wo_loc)

    out = pl.pallas_call(
        _allreduce_body,
        out_shape=jax.ShapeDtypeStruct((SQ, D_MODEL), jnp.float32),
        in_specs=[pl.BlockSpec(memory_space=pltpu.VMEM)],
        out_specs=pl.BlockSpec(memory_space=pltpu.VMEM),
        scratch_shapes=[
            pltpu.VMEM((6 * CHUNK, D_MODEL), jnp.float32),
            pltpu.SemaphoreType.DMA((12,)),
            pltpu.SemaphoreType.DMA((12,)),
        ],
        compiler_params=pltpu.CompilerParams(
            collective_id=0, vmem_limit_bytes=100 * 1024 * 1024
        ),
    )(partial)

    return out.reshape(1, SQ, D_MODEL)


# device time: 190972 ns/iter; 1.4188x vs baseline; 1.0803x over previous
import jax
import jax.numpy as jnp
from jax import lax
from jax.experimental import pallas as pl
from jax.experimental.pallas import tpu as pltpu

N_DEV = 4
SQ = 2048
SKV = 2048
D_MODEL = 1024
H_LOC = 8
DH = 128
BLK = 64
SCALE = 0.08838834764831843
QT = 512
N_QT = SQ // QT
CHUNK = SQ // (2 * N_DEV)


def _compute_body(x_ref, wq_ref, k_ref, v_ref, wo_ref, out_ref):
    h = pl.program_id(0)
    q = jnp.dot(x_ref[...], wq_ref[...], preferred_element_type=jnp.float32)
    k = k_ref[...]
    v = v_ref[...]
    ctxs = []
    for qt in range(N_QT):
        qq = q[qt * QT:(qt + 1) * QT, :]
        sd = lax.dot_general(
            qq, k[qt * QT:(qt + 1) * QT, :], (((1,), (1,)), ((), ())),
            preferred_element_type=jnp.float32,
        ) * SCALE
        row = lax.broadcasted_iota(jnp.int32, (QT, QT), 0)
        col = lax.broadcasted_iota(jnp.int32, (QT, QT), 1)
        wd = jnp.where((col // BLK) <= (row // BLK), jnp.exp(sd), 0.0)
        denom = jnp.sum(wd, axis=-1, keepdims=True)
        ctx_u = jnp.dot(
            wd, v[qt * QT:(qt + 1) * QT, :], preferred_element_type=jnp.float32
        )
        if qt > 0:
            so = lax.dot_general(
                qq, k[:qt * QT, :], (((1,), (1,)), ((), ())),
                preferred_element_type=jnp.float32,
            ) * SCALE
            wo_ = jnp.exp(so)
            denom = denom + jnp.sum(wo_, axis=-1, keepdims=True)
            ctx_u = ctx_u + jnp.dot(
                wo_, v[:qt * QT, :], preferred_element_type=jnp.float32
            )
        ctxs.append(ctx_u / denom)
    ctx = jnp.concatenate(ctxs, axis=0)
    contrib = jnp.dot(ctx, wo_ref[...], preferred_element_type=jnp.float32)

    @pl.when(h == 0)
    def _():
        out_ref[...] = contrib

    @pl.when(h != 0)
    def _():
        out_ref[...] += contrib


def _allreduce_body(p_ref, out_ref, comm_ref, send_sems, recv_sems):
    my = lax.axis_index("i")
    left = lax.rem(my + (N_DEV - 1), N_DEV)
    right = lax.rem(my + 1, N_DEV)

    def mod4(v):
        return lax.rem(v + 4 * N_DEV, N_DEV)

    def rows_r(c):
        return c * CHUNK

    def rows_l(c):
        return N_DEV * CHUNK + c * CHUNK

    barrier_sem = pltpu.get_barrier_semaphore()
    for nbr in (left, right):
        pl.semaphore_signal(
            barrier_sem, inc=1,
            device_id=(nbr,), device_id_type=pl.DeviceIdType.MESH,
        )
    pl.semaphore_wait(barrier_sem, 2)

    out_ref[...] = p_ref[...]

    def copy(src_start, dst_start, dst_is_out, dev, sem_idx):
        dst = out_ref if dst_is_out else comm_ref
        return pltpu.make_async_remote_copy(
            src_ref=out_ref.at[pl.ds(src_start, CHUNK), :],
            dst_ref=dst.at[pl.ds(dst_start, CHUNK), :],
            send_sem=send_sems.at[sem_idx],
            recv_sem=recv_sems.at[sem_idx],
            device_id=(dev,),
            device_id_type=pl.DeviceIdType.MESH,
        )

    for s in range(N_DEV - 1):
        r_send = copy(rows_r(mod4(my - s)), s * CHUNK, False, right, s)
        l_send = copy(rows_l(mod4(my + s)), (3 + s) * CHUNK, False, left, 3 + s)
        r_send.start()
        l_send.start()
        r_send.wait()
        l_send.wait()
        rr = rows_r(mod4(my - s - 1))
        rl = rows_l(mod4(my + s + 1))
        out_ref[pl.ds(rr, CHUNK), :] += comm_ref[pl.ds(s * CHUNK, CHUNK), :]
        out_ref[pl.ds(rl, CHUNK), :] += comm_ref[pl.ds((3 + s) * CHUNK, CHUNK), :]

    for s in range(N_DEV - 1):
        cr = rows_r(mod4(my + 1 - s))
        cl = rows_l(mod4(my - 1 + s))
        r_send = copy(cr, cr, True, right, 6 + s)
        l_send = copy(cl, cl, True, left, 9 + s)
        r_send.start()
        l_send.start()
        r_send.wait()
        l_send.wait()


def kernel(x, Wq, K_ext, V_ext, Wo):
    i = lax.axis_index("i")
    Wq_loc = lax.dynamic_slice(Wq, (0, i * (H_LOC * DH)), (D_MODEL, H_LOC * DH))
    Wo_loc = lax.dynamic_slice(Wo, (i * (H_LOC * DH), 0), (H_LOC * DH, D_MODEL))
    x2 = x.reshape(SQ, D_MODEL)
    K = K_ext.reshape(SKV, H_LOC * DH)
    V = V_ext.reshape(SKV, H_LOC * DH)

    partial = pl.pallas_call(
        _compute_body,
        grid=(H_LOC,),
        in_specs=[
            pl.BlockSpec((SQ, D_MODEL), lambda h: (0, 0)),
            pl.BlockSpec((D_MODEL, DH), lambda h: (0, h)),
            pl.BlockSpec((SKV, DH), lambda h: (0, h)),
            pl.BlockSpec((SKV, DH), lambda h: (0, h)),
            pl.BlockSpec((DH, D_MODEL), lambda h: (h, 0)),
        ],
        out_specs=pl.BlockSpec((SQ, D_MODEL), lambda h: (0, 0)),
        out_shape=jax.ShapeDtypeStruct((SQ, D_MODEL), jnp.float32),
        compiler_params=pltpu.CompilerParams(
            vmem_limit_bytes=100 * 1024 * 1024
        ),
    )(x2, Wq_loc, K, V, Wo_loc)

    out = pl.pallas_call(
        _allreduce_body,
        out_shape=jax.ShapeDtypeStruct((SQ, D_MODEL), jnp.float32),
        in_specs=[pl.BlockSpec(memory_space=pltpu.VMEM)],
        out_specs=pl.BlockSpec(memory_space=pltpu.VMEM),
        scratch_shapes=[
            pltpu.VMEM((6 * CHUNK, D_MODEL), jnp.float32),
            pltpu.SemaphoreType.DMA((12,)),
            pltpu.SemaphoreType.DMA((12,)),
        ],
        compiler_params=pltpu.CompilerParams(
            collective_id=0, vmem_limit_bytes=100 * 1024 * 1024
        ),
    )(partial)

    return out.reshape(1, SQ, D_MODEL)


# device time: 186158 ns/iter; 1.4555x vs baseline; 1.0259x over previous
import jax
import jax.numpy as jnp
from jax import lax
from jax.experimental import pallas as pl
from jax.experimental.pallas import tpu as pltpu

N_DEV = 4
SQ = 2048
SKV = 2048
D_MODEL = 1024
H_LOC = 8
DH = 128
BLK = 64
SCALE = 0.08838834764831843
QT = 512
N_QT = SQ // QT
CHUNK = SQ // (2 * N_DEV)


def _compute_body(
    x_ref, wq_hbm, k_ref, v_ref, wo_hbm, out_ref, wq_s, wo_s, load_sems
):
    h = pl.program_id(0)
    my = lax.axis_index("i")

    @pl.when(h == 0)
    def _():
        cpq = pltpu.make_async_copy(
            wq_hbm.at[:, pl.ds(my * (H_LOC * DH), H_LOC * DH)],
            wq_s, load_sems.at[0],
        )
        cpo = pltpu.make_async_copy(
            wo_hbm.at[pl.ds(my * (H_LOC * DH), H_LOC * DH), :],
            wo_s, load_sems.at[1],
        )
        cpq.start()
        cpo.start()
        cpq.wait()
        cpo.wait()

    q = jnp.dot(
        x_ref[...], wq_s[:, pl.ds(h * DH, DH)],
        preferred_element_type=jnp.float32,
    )
    k = k_ref[...]
    v = v_ref[...]
    ctxs = []
    for qt in range(N_QT):
        qq = q[qt * QT:(qt + 1) * QT, :]
        sd = lax.dot_general(
            qq, k[qt * QT:(qt + 1) * QT, :], (((1,), (1,)), ((), ())),
            preferred_element_type=jnp.float32,
        ) * SCALE
        row = lax.broadcasted_iota(jnp.int32, (QT, QT), 0)
        col = lax.broadcasted_iota(jnp.int32, (QT, QT), 1)
        wd = jnp.where((col // BLK) <= (row // BLK), jnp.exp(sd), 0.0)
        denom = jnp.sum(wd, axis=-1, keepdims=True)
        ctx_u = jnp.dot(
            wd, v[qt * QT:(qt + 1) * QT, :], preferred_element_type=jnp.float32
        )
        if qt > 0:
            so = lax.dot_general(
                qq, k[:qt * QT, :], (((1,), (1,)), ((), ())),
                preferred_element_type=jnp.float32,
            ) * SCALE
            wo_ = jnp.exp(so)
            denom = denom + jnp.sum(wo_, axis=-1, keepdims=True)
            ctx_u = ctx_u + jnp.dot(
                wo_, v[:qt * QT, :], preferred_element_type=jnp.float32
            )
        ctxs.append(ctx_u / denom)
    ctx = jnp.concatenate(ctxs, axis=0)
    contrib = jnp.dot(
        ctx, wo_s[pl.ds(h * DH, DH), :], preferred_element_type=jnp.float32
    )

    @pl.when(h == 0)
    def _():
        out_ref[...] = contrib

    @pl.when(h != 0)
    def _():
        out_ref[...] += contrib


def _allreduce_body(p_ref, out_ref, comm_ref, p_sems, send_sems, recv_sems):
    my = lax.axis_index("i")
    left = lax.rem(my + (N_DEV - 1), N_DEV)
    right = lax.rem(my + 1, N_DEV)

    def mod4(v):
        return lax.rem(v + 4 * N_DEV, N_DEV)

    def rows_r(c):
        return c * CHUNK

    def rows_l(c):
        return N_DEV * CHUNK + c * CHUNK

    barrier_sem = pltpu.get_barrier_semaphore()
    for nbr in (left, right):
        pl.semaphore_signal(
            barrier_sem, inc=1,
            device_id=(nbr,), device_id_type=pl.DeviceIdType.MESH,
        )
    pl.semaphore_wait(barrier_sem, 2)

    p_copies = []
    for step in range(N_DEV):
        for start in (rows_r(mod4(my - step)), rows_l(mod4(my + step))):
            cp = pltpu.make_async_copy(
                p_ref.at[pl.ds(start, CHUNK), :],
                out_ref.at[pl.ds(start, CHUNK), :],
                p_sems.at[len(p_copies)],
            )
            cp.start()
            p_copies.append(cp)

    def copy(src_start, dst_start, dst_is_out, dev, sem_idx):
        dst = out_ref if dst_is_out else comm_ref
        return pltpu.make_async_remote_copy(
            src_ref=out_ref.at[pl.ds(src_start, CHUNK), :],
            dst_ref=dst.at[pl.ds(dst_start, CHUNK), :],
            send_sem=send_sems.at[sem_idx],
            recv_sem=recv_sems.at[sem_idx],
            device_id=(dev,),
            device_id_type=pl.DeviceIdType.MESH,
        )

    for s in range(N_DEV - 1):
        if s == 0:
            p_copies[0].wait()
            p_copies[1].wait()
        r_send = copy(rows_r(mod4(my - s)), s * CHUNK, False, right, s)
        l_send = copy(rows_l(mod4(my + s)), (3 + s) * CHUNK, False, left, 3 + s)
        r_send.start()
        l_send.start()
        p_copies[2 * s + 2].wait()
        p_copies[2 * s + 3].wait()
        r_send.wait()
        l_send.wait()
        rr = rows_r(mod4(my - s - 1))
        rl = rows_l(mod4(my + s + 1))
        out_ref[pl.ds(rr, CHUNK), :] += comm_ref[pl.ds(s * CHUNK, CHUNK), :]
        out_ref[pl.ds(rl, CHUNK), :] += comm_ref[pl.ds((3 + s) * CHUNK, CHUNK), :]

    for s in range(N_DEV - 1):
        cr = rows_r(mod4(my + 1 - s))
        cl = rows_l(mod4(my - 1 + s))
        r_send = copy(cr, cr, True, right, 6 + s)
        l_send = copy(cl, cl, True, left, 9 + s)
        r_send.start()
        l_send.start()
        r_send.wait()
        l_send.wait()


def kernel(x, Wq, K_ext, V_ext, Wo):
    i = lax.axis_index("i")
    x2 = x.reshape(SQ, D_MODEL)
    K = K_ext.reshape(SKV, H_LOC * DH)
    V = V_ext.reshape(SKV, H_LOC * DH)

    partial = pl.pallas_call(
        _compute_body,
        grid=(H_LOC,),
        in_specs=[
            pl.BlockSpec((SQ, D_MODEL), lambda h: (0, 0)),
            pl.BlockSpec(memory_space=pl.ANY),
            pl.BlockSpec((SKV, DH), lambda h: (0, h)),
            pl.BlockSpec((SKV, DH), lambda h: (0, h)),
            pl.BlockSpec(memory_space=pl.ANY),
        ],
        out_specs=pl.BlockSpec((SQ, D_MODEL), lambda h: (0, 0)),
        out_shape=jax.ShapeDtypeStruct((SQ, D_MODEL), jnp.float32),
        scratch_shapes=[
            pltpu.VMEM((D_MODEL, H_LOC * DH), jnp.float32),
            pltpu.VMEM((H_LOC * DH, D_MODEL), jnp.float32),
            pltpu.SemaphoreType.DMA((2,)),
        ],
        compiler_params=pltpu.CompilerParams(
            vmem_limit_bytes=100 * 1024 * 1024
        ),
    )(x2, Wq, K, V, Wo)

    out = pl.pallas_call(
        _allreduce_body,
        out_shape=jax.ShapeDtypeStruct((SQ, D_MODEL), jnp.float32),
        in_specs=[pl.BlockSpec(memory_space=pl.ANY)],
        out_specs=pl.BlockSpec(memory_space=pltpu.VMEM),
        scratch_shapes=[
            pltpu.VMEM((6 * CHUNK, D_MODEL), jnp.float32),
            pltpu.SemaphoreType.DMA((8,)),
            pltpu.SemaphoreType.DMA((12,)),
            pltpu.SemaphoreType.DMA((12,)),
        ],
        compiler_params=pltpu.CompilerParams(
            collective_id=0, vmem_limit_bytes=100 * 1024 * 1024
        ),
    )(partial)

    return out.reshape(1, SQ, D_MODEL)


# device time: 185609 ns/iter; 1.4598x vs baseline; 1.0030x over previous
import jax
import jax.numpy as jnp
from jax import lax
from jax.experimental import pallas as pl
from jax.experimental.pallas import tpu as pltpu

N_DEV = 4
SQ = 2048
SKV = 2048
D_MODEL = 1024
H_LOC = 8
DH = 128
BLK = 64
SCALE = 0.08838834764831843
QT = 512
N_QT = SQ // QT
CHUNK = SQ // (2 * N_DEV)


def _compute_body(
    x_ref, wq_hbm, k_ref, v_ref, wo_hbm, out_ref, wq_s, wo_s, load_sems
):
    h = pl.program_id(0)
    my = lax.axis_index("i")

    @pl.when(h == 0)
    def _():
        cpq = pltpu.make_async_copy(
            wq_hbm.at[:, pl.ds(my * (H_LOC * DH), H_LOC * DH)],
            wq_s, load_sems.at[0],
        )
        cpo = pltpu.make_async_copy(
            wo_hbm.at[pl.ds(my * (H_LOC * DH), H_LOC * DH), :],
            wo_s, load_sems.at[1],
        )
        cpq.start()
        cpo.start()
        cpq.wait()
        cpo.wait()

    q = jnp.dot(
        x_ref[...], wq_s[:, pl.ds(h * DH, DH)],
        preferred_element_type=jnp.float32,
    )
    qb = q.astype(jnp.bfloat16)
    k = k_ref[...].astype(jnp.bfloat16)
    v = v_ref[...].astype(jnp.bfloat16)
    ctxs = []
    for qt in range(N_QT):
        qq = qb[qt * QT:(qt + 1) * QT, :]
        sd = lax.dot_general(
            qq, k[qt * QT:(qt + 1) * QT, :], (((1,), (1,)), ((), ())),
            preferred_element_type=jnp.float32,
        ) * SCALE
        row = lax.broadcasted_iota(jnp.int32, (QT, QT), 0)
        col = lax.broadcasted_iota(jnp.int32, (QT, QT), 1)
        wd = jnp.where((col // BLK) <= (row // BLK), jnp.exp(sd), 0.0)
        denom = jnp.sum(wd, axis=-1, keepdims=True)
        ctx_u = jnp.dot(
            wd.astype(jnp.bfloat16), v[qt * QT:(qt + 1) * QT, :],
            preferred_element_type=jnp.float32,
        )
        if qt > 0:
            so = lax.dot_general(
                qq, k[:qt * QT, :], (((1,), (1,)), ((), ())),
                preferred_element_type=jnp.float32,
            ) * SCALE
            wo_ = jnp.exp(so)
            denom = denom + jnp.sum(wo_, axis=-1, keepdims=True)
            ctx_u = ctx_u + jnp.dot(
                wo_.astype(jnp.bfloat16), v[:qt * QT, :],
                preferred_element_type=jnp.float32,
            )
        ctxs.append(ctx_u / denom)
    ctx = jnp.concatenate(ctxs, axis=0)
    contrib = jnp.dot(
        ctx, wo_s[pl.ds(h * DH, DH), :], preferred_element_type=jnp.float32
    )

    @pl.when(h == 0)
    def _():
        out_ref[...] = contrib

    @pl.when(h != 0)
    def _():
        out_ref[...] += contrib


def _allreduce_body(p_ref, out_ref, comm_ref, p_sems, send_sems, recv_sems):
    my = lax.axis_index("i")
    left = lax.rem(my + (N_DEV - 1), N_DEV)
    right = lax.rem(my + 1, N_DEV)

    def mod4(v):
        return lax.rem(v + 4 * N_DEV, N_DEV)

    def rows_r(c):
        return c * CHUNK

    def rows_l(c):
        return N_DEV * CHUNK + c * CHUNK

    barrier_sem = pltpu.get_barrier_semaphore()
    for nbr in (left, right):
        pl.semaphore_signal(
            barrier_sem, inc=1,
            device_id=(nbr,), device_id_type=pl.DeviceIdType.MESH,
        )
    pl.semaphore_wait(barrier_sem, 2)

    p_copies = []
    for step in range(N_DEV):
        for start in (rows_r(mod4(my - step)), rows_l(mod4(my + step))):
            cp = pltpu.make_async_copy(
                p_ref.at[pl.ds(start, CHUNK), :],
                out_ref.at[pl.ds(start, CHUNK), :],
                p_sems.at[len(p_copies)],
            )
            cp.start()
            p_copies.append(cp)

    def copy(src_start, dst_start, dst_is_out, dev, sem_idx):
        dst = out_ref if dst_is_out else comm_ref
        return pltpu.make_async_remote_copy(
            src_ref=out_ref.at[pl.ds(src_start, CHUNK), :],
            dst_ref=dst.at[pl.ds(dst_start, CHUNK), :],
            send_sem=send_sems.at[sem_idx],
            recv_sem=recv_sems.at[sem_idx],
            device_id=(dev,),
            device_id_type=pl.DeviceIdType.MESH,
        )

    for s in range(N_DEV - 1):
        if s == 0:
            p_copies[0].wait()
            p_copies[1].wait()
        r_send = copy(rows_r(mod4(my - s)), s * CHUNK, False, right, s)
        l_send = copy(rows_l(mod4(my + s)), (3 + s) * CHUNK, False, left, 3 + s)
        r_send.start()
        l_send.start()
        p_copies[2 * s + 2].wait()
        p_copies[2 * s + 3].wait()
        r_send.wait()
        l_send.wait()
        rr = rows_r(mod4(my - s - 1))
        rl = rows_l(mod4(my + s + 1))
        out_ref[pl.ds(rr, CHUNK), :] += comm_ref[pl.ds(s * CHUNK, CHUNK), :]
        out_ref[pl.ds(rl, CHUNK), :] += comm_ref[pl.ds((3 + s) * CHUNK, CHUNK), :]

    for s in range(N_DEV - 1):
        cr = rows_r(mod4(my + 1 - s))
        cl = rows_l(mod4(my - 1 + s))
        r_send = copy(cr, cr, True, right, 6 + s)
        l_send = copy(cl, cl, True, left, 9 + s)
        r_send.start()
        l_send.start()
        r_send.wait()
        l_send.wait()


def kernel(x, Wq, K_ext, V_ext, Wo):
    i = lax.axis_index("i")
    x2 = x.reshape(SQ, D_MODEL)
    K = K_ext.reshape(SKV, H_LOC * DH)
    V = V_ext.reshape(SKV, H_LOC * DH)

    partial = pl.pallas_call(
        _compute_body,
        grid=(H_LOC,),
        in_specs=[
            pl.BlockSpec((SQ, D_MODEL), lambda h: (0, 0)),
            pl.BlockSpec(memory_space=pl.ANY),
            pl.BlockSpec((SKV, DH), lambda h: (0, h)),
            pl.BlockSpec((SKV, DH), lambda h: (0, h)),
            pl.BlockSpec(memory_space=pl.ANY),
        ],
        out_specs=pl.BlockSpec((SQ, D_MODEL), lambda h: (0, 0)),
        out_shape=jax.ShapeDtypeStruct((SQ, D_MODEL), jnp.float32),
        scratch_shapes=[
            pltpu.VMEM((D_MODEL, H_LOC * DH), jnp.float32),
            pltpu.VMEM((H_LOC * DH, D_MODEL), jnp.float32),
            pltpu.SemaphoreType.DMA((2,)),
        ],
        compiler_params=pltpu.CompilerParams(
            vmem_limit_bytes=100 * 1024 * 1024
        ),
    )(x2, Wq, K, V, Wo)

    out = pl.pallas_call(
        _allreduce_body,
        out_shape=jax.ShapeDtypeStruct((SQ, D_MODEL), jnp.float32),
        in_specs=[pl.BlockSpec(memory_space=pl.ANY)],
        out_specs=pl.BlockSpec(memory_space=pltpu.VMEM),
        scratch_shapes=[
            pltpu.VMEM((6 * CHUNK, D_MODEL), jnp.float32),
            pltpu.SemaphoreType.DMA((8,)),
            pltpu.SemaphoreType.DMA((12,)),
            pltpu.SemaphoreType.DMA((12,)),
        ],
        compiler_params=pltpu.CompilerParams(
            collective_id=0, vmem_limit_bytes=100 * 1024 * 1024
        ),
    )(partial)

    return out.reshape(1, SQ, D_MODEL)


# device time: 146368 ns/iter; 1.8511x vs baseline; 1.2681x over previous
import jax
import jax.numpy as jnp
from jax import lax
from jax.experimental import pallas as pl
from jax.experimental.pallas import tpu as pltpu

N_DEV = 4
SQ = 2048
SKV = 2048
D_MODEL = 1024
H_LOC = 8
DH = 128
BLK = 64
SCALE = 0.08838834764831843
QT = 512
N_QT = SQ // QT
CHUNK = SQ // (2 * N_DEV)


def _compute_body(
    x_ref, wq_hbm, k_ref, v_ref, wo_hbm, out_ref, wq_s, wo_s, load_sems
):
    h = pl.program_id(0)
    my = lax.axis_index("i")

    @pl.when(h == 0)
    def _():
        cpq = pltpu.make_async_copy(
            wq_hbm.at[:, pl.ds(my * (H_LOC * DH), H_LOC * DH)],
            wq_s, load_sems.at[0],
        )
        cpo = pltpu.make_async_copy(
            wo_hbm.at[pl.ds(my * (H_LOC * DH), H_LOC * DH), :],
            wo_s, load_sems.at[1],
        )
        cpq.start()
        cpo.start()
        cpq.wait()
        cpo.wait()

    q = jnp.dot(
        x_ref[...], wq_s[:, pl.ds(h * DH, DH)],
        preferred_element_type=jnp.float32,
    )
    qb = (q * SCALE).astype(jnp.bfloat16)
    k = k_ref[...].astype(jnp.bfloat16)
    v = v_ref[...].astype(jnp.bfloat16)
    ctxs = []
    for qt in range(N_QT):
        qq = qb[qt * QT:(qt + 1) * QT, :]
        sd = lax.dot_general(
            qq, k[qt * QT:(qt + 1) * QT, :], (((1,), (1,)), ((), ())),
            preferred_element_type=jnp.float32,
        )
        row = lax.broadcasted_iota(jnp.int32, (QT, QT), 0)
        col = lax.broadcasted_iota(jnp.int32, (QT, QT), 1)
        wd = jnp.where((col // BLK) <= (row // BLK), jnp.exp(sd), 0.0)
        denom = jnp.sum(wd, axis=-1, keepdims=True)
        ctx_u = jnp.dot(
            wd.astype(jnp.bfloat16), v[qt * QT:(qt + 1) * QT, :],
            preferred_element_type=jnp.float32,
        )
        if qt > 0:
            so = lax.dot_general(
                qq, k[:qt * QT, :], (((1,), (1,)), ((), ())),
                preferred_element_type=jnp.float32,
            )
            wo_ = jnp.exp(so)
            denom = denom + jnp.sum(wo_, axis=-1, keepdims=True)
            ctx_u = ctx_u + jnp.dot(
                wo_.astype(jnp.bfloat16), v[:qt * QT, :],
                preferred_element_type=jnp.float32,
            )
        ctxs.append(ctx_u / denom)
    ctx = jnp.concatenate(ctxs, axis=0)
    contrib = jnp.dot(
        ctx, wo_s[pl.ds(h * DH, DH), :], preferred_element_type=jnp.float32
    )

    @pl.when(h == 0)
    def _():
        out_ref[...] = contrib.astype(jnp.bfloat16)

    @pl.when(h != 0)
    def _():
        out_ref[...] += contrib.astype(jnp.bfloat16)


def _allreduce_body(p_ref, out_ref, comm_ref, p_sems, send_sems, recv_sems):
    my = lax.axis_index("i")
    left = lax.rem(my + (N_DEV - 1), N_DEV)
    right = lax.rem(my + 1, N_DEV)

    def mod4(v):
        return lax.rem(v + 4 * N_DEV, N_DEV)

    def rows_r(c):
        return c * CHUNK

    def rows_l(c):
        return N_DEV * CHUNK + c * CHUNK

    barrier_sem = pltpu.get_barrier_semaphore()
    for nbr in (left, right):
        pl.semaphore_signal(
            barrier_sem, inc=1,
            device_id=(nbr,), device_id_type=pl.DeviceIdType.MESH,
        )
    pl.semaphore_wait(barrier_sem, 2)

    p_copies = []
    for step in range(N_DEV):
        for start in (rows_r(mod4(my - step)), rows_l(mod4(my + step))):
            cp = pltpu.make_async_copy(
                p_ref.at[pl.ds(start, CHUNK), :],
                out_ref.at[pl.ds(start, CHUNK), :],
                p_sems.at[len(p_copies)],
            )
            cp.start()
            p_copies.append(cp)

    def copy(src_start, dst_start, dst_is_out, dev, sem_idx):
        dst = out_ref if dst_is_out else comm_ref
        return pltpu.make_async_remote_copy(
            src_ref=out_ref.at[pl.ds(src_start, CHUNK), :],
            dst_ref=dst.at[pl.ds(dst_start, CHUNK), :],
            send_sem=send_sems.at[sem_idx],
            recv_sem=recv_sems.at[sem_idx],
            device_id=(dev,),
            device_id_type=pl.DeviceIdType.MESH,
        )

    for s in range(N_DEV - 1):
        if s == 0:
            p_copies[0].wait()
            p_copies[1].wait()
        r_send = copy(rows_r(mod4(my - s)), s * CHUNK, False, right, s)
        l_send = copy(rows_l(mod4(my + s)), (3 + s) * CHUNK, False, left, 3 + s)
        r_send.start()
        l_send.start()
        p_copies[2 * s + 2].wait()
        p_copies[2 * s + 3].wait()
        r_send.wait()
        l_send.wait()
        rr = rows_r(mod4(my - s - 1))
        rl = rows_l(mod4(my + s + 1))
        out_ref[pl.ds(rr, CHUNK), :] += comm_ref[pl.ds(s * CHUNK, CHUNK), :]
        out_ref[pl.ds(rl, CHUNK), :] += comm_ref[pl.ds((3 + s) * CHUNK, CHUNK), :]

    for s in range(N_DEV - 1):
        cr = rows_r(mod4(my + 1 - s))
        cl = rows_l(mod4(my - 1 + s))
        r_send = copy(cr, cr, True, right, 6 + s)
        l_send = copy(cl, cl, True, left, 9 + s)
        r_send.start()
        l_send.start()
        r_send.wait()
        l_send.wait()


def kernel(x, Wq, K_ext, V_ext, Wo):
    i = lax.axis_index("i")
    x2 = x.reshape(SQ, D_MODEL)
    K = K_ext.reshape(SKV, H_LOC * DH)
    V = V_ext.reshape(SKV, H_LOC * DH)

    partial = pl.pallas_call(
        _compute_body,
        grid=(H_LOC,),
        in_specs=[
            pl.BlockSpec((SQ, D_MODEL), lambda h: (0, 0)),
            pl.BlockSpec(memory_space=pl.ANY),
            pl.BlockSpec((SKV, DH), lambda h: (0, h)),
            pl.BlockSpec((SKV, DH), lambda h: (0, h)),
            pl.BlockSpec(memory_space=pl.ANY),
        ],
        out_specs=pl.BlockSpec((SQ, D_MODEL), lambda h: (0, 0)),
        out_shape=jax.ShapeDtypeStruct((SQ, D_MODEL), jnp.bfloat16),
        scratch_shapes=[
            pltpu.VMEM((D_MODEL, H_LOC * DH), jnp.float32),
            pltpu.VMEM((H_LOC * DH, D_MODEL), jnp.float32),
            pltpu.SemaphoreType.DMA((2,)),
        ],
        compiler_params=pltpu.CompilerParams(
            vmem_limit_bytes=100 * 1024 * 1024
        ),
    )(x2, Wq, K, V, Wo)

    out = pl.pallas_call(
        _allreduce_body,
        out_shape=jax.ShapeDtypeStruct((SQ, D_MODEL), jnp.bfloat16),
        in_specs=[pl.BlockSpec(memory_space=pl.ANY)],
        out_specs=pl.BlockSpec(memory_space=pltpu.VMEM),
        scratch_shapes=[
            pltpu.VMEM((6 * CHUNK, D_MODEL), jnp.bfloat16),
            pltpu.SemaphoreType.DMA((8,)),
            pltpu.SemaphoreType.DMA((12,)),
            pltpu.SemaphoreType.DMA((12,)),
        ],
        compiler_params=pltpu.CompilerParams(
            collective_id=0, vmem_limit_bytes=100 * 1024 * 1024
        ),
    )(partial)

    return out.astype(jnp.float32).reshape(1, SQ, D_MODEL)
